# Initial kernel scaffold; baseline (speedup 1.0000x reference)
import jax
import jax.numpy as jnp
from jax.experimental import pallas as pl


def kernel(rpn_bbox_deltas, rpn_probs, anchors):
    raise NotImplementedError("write your pallas kernel here")


# speedup vs baseline: 11.8230x; 11.8230x over previous
import functools

import jax
import jax.numpy as jnp
from jax import lax
from jax.experimental import pallas as pl
from jax.experimental.pallas import tpu as pltpu
from jax.experimental.pallas import tpu_sc as plsc

B = 4
A = 8649
AP = 8704
N = 6000
NO = 1500
NOP = 1504
L = 16
NCHUNK = N // L
IOU_T = 0.7
V0, V1, V2, V3 = 0.1, 0.1, 0.2, 0.2

_mesh = plsc.VectorSubcoreMesh(
    core_axis_name="c", subcore_axis_name="s", num_cores=2, num_subcores=16)


@functools.partial(
    pl.kernel,
    out_type=(
        jax.ShapeDtypeStruct((B, N), jnp.float32),
        jax.ShapeDtypeStruct((B, NOP), jnp.float32),
    ),
    mesh=_mesh,
    compiler_params=pltpu.CompilerParams(needs_layout_passes=False),
    scratch_types=[
        pltpu.VMEM((AP,), jnp.float32),
        pltpu.VMEM((AP,), jnp.float32),
        pltpu.VMEM((AP,), jnp.float32),
        pltpu.VMEM((AP,), jnp.float32),
        pltpu.VMEM((AP,), jnp.float32),
        pltpu.VMEM((AP,), jnp.float32),
        pltpu.VMEM((AP,), jnp.float32),
        pltpu.VMEM((AP,), jnp.float32),
        pltpu.VMEM((N,), jnp.int32),
        pltpu.VMEM((N,), jnp.float32),
        pltpu.VMEM((N,), jnp.float32),
        pltpu.VMEM((N,), jnp.float32),
        pltpu.VMEM((N,), jnp.float32),
        pltpu.VMEM((N,), jnp.float32),
        pltpu.VMEM((N,), jnp.float32),
        pltpu.VMEM((N,), jnp.int32),
        pltpu.VMEM((N,), jnp.float32),
        pltpu.VMEM((NOP,), jnp.float32),
    ],
)
def _sc_nms(dts_hbm, anc_hbm, sc_hbm, idx_hbm, rois_hbm, rsc_hbm,
            d0v, d1v, d2v, d3v, a0v, a1v, a2v, a3v,
            iv, sv, y1v, x1v, y2v, x2v, arv, kv, obv, osv):
    w = lax.axis_index("s") * 2 + lax.axis_index("c")

    @pl.when(w < B)
    def _body():
        pltpu.sync_copy(dts_hbm.at[w * 4], d0v)
        pltpu.sync_copy(dts_hbm.at[w * 4 + 1], d1v)
        pltpu.sync_copy(dts_hbm.at[w * 4 + 2], d2v)
        pltpu.sync_copy(dts_hbm.at[w * 4 + 3], d3v)
        pltpu.sync_copy(anc_hbm.at[0], a0v)
        pltpu.sync_copy(anc_hbm.at[1], a1v)
        pltpu.sync_copy(anc_hbm.at[2], a2v)
        pltpu.sync_copy(anc_hbm.at[3], a3v)
        pltpu.sync_copy(sc_hbm.at[w], sv)
        pltpu.sync_copy(idx_hbm.at[w], iv)

        lane = lax.iota(jnp.int32, L)

        def gather_body(ci, carry):
            s = pl.ds(ci * L, L)
            ix = iv[s]
            d0 = plsc.load_gather(d0v, [ix])
            d1 = plsc.load_gather(d1v, [ix])
            d2 = plsc.load_gather(d2v, [ix])
            d3 = plsc.load_gather(d3v, [ix])
            ay1 = plsc.load_gather(a0v, [ix])
            ax1 = plsc.load_gather(a1v, [ix])
            ay2 = plsc.load_gather(a2v, [ix])
            ax2 = plsc.load_gather(a3v, [ix])
            aw = ax2 - ax1
            ah = ay2 - ay1
            acx = ax1 + 0.5 * aw
            acy = ay1 + 0.5 * ah
            bw = jnp.exp(d3 * V3) * aw
            bh = jnp.exp(d2 * V2) * ah
            bcx = d1 * V1 * aw + acx
            bcy = d0 * V0 * ah + acy
            yy1 = jnp.clip(bcy - 0.5 * bh, 0.0, 1.0)
            xx1 = jnp.clip(bcx - 0.5 * bw, 0.0, 1.0)
            yy2 = jnp.clip(bh + (bcy - 0.5 * bh), 0.0, 1.0)
            xx2 = jnp.clip(bw + (bcx - 0.5 * bw), 0.0, 1.0)
            y1v[s] = yy1
            x1v[s] = xx1
            y2v[s] = yy2
            x2v[s] = xx2
            arv[s] = (yy2 - yy1) * (xx2 - xx1)
            kv[s] = jnp.ones((L,), jnp.int32)
            return carry

        lax.fori_loop(0, NCHUNK, gather_body, 0)

        def row_body(i, carry):
            spi = jnp.full((L,), i, jnp.int32)
            alive = plsc.load_gather(kv, [spi])

            @pl.when(alive[0] == 1)
            def _row():
                by1 = plsc.load_gather(y1v, [spi])
                bx1 = plsc.load_gather(x1v, [spi])
                by2 = plsc.load_gather(y2v, [spi])
                bx2 = plsc.load_gather(x2v, [spi])
                bar = plsc.load_gather(arv, [spi])

                def chunk_body(c, carry2):
                    s = pl.ds(c * L, L)
                    iy1 = jnp.maximum(y1v[s], by1)
                    ix1 = jnp.maximum(x1v[s], bx1)
                    iy2 = jnp.minimum(y2v[s], by2)
                    ix2 = jnp.minimum(x2v[s], bx2)
                    inter = (jnp.maximum(iy2 - iy1, 0.0)
                             * jnp.maximum(ix2 - ix1, 0.0))
                    denom = arv[s] + bar - inter + 1e-9
                    jidx = c * L + lane
                    supp = (inter > IOU_T * denom) & (jidx > i)
                    kv[s] = jnp.where(supp, 0, kv[s])
                    return carry2

                lax.fori_loop(i // L, NCHUNK, chunk_body, 0)
            return carry

        lax.fori_loop(0, N, row_body, 0)

        def zero_body(ci, carry):
            obv[pl.ds(ci * L, L)] = jnp.zeros((L,), jnp.float32)
            return carry

        lax.fori_loop(0, NCHUNK, zero_body, 0)

        def zero2_body(ci, carry):
            osv[pl.ds(ci * L, L)] = jnp.zeros((L,), jnp.float32)
            return carry

        lax.fori_loop(0, NOP // L, zero2_body, 0)

        def comp_body(ci, base):
            s = pl.ds(ci * L, L)
            kb = kv[s] == 1
            kbi = jnp.where(kb, 1, 0)
            pos = plsc.cumsum(kbi)
            slot = base + pos - 1
            wm = kb & (slot < NO)
            scs = sv[s]
            validf = jnp.where(scs > 0.0, 1.0, 0.0)
            plsc.store_scatter(osv, [slot], scs * validf, mask=wm)
            s4 = slot * 4
            plsc.store_scatter(obv, [s4], y1v[s] * validf, mask=wm)
            plsc.store_scatter(obv, [s4 + 1], x1v[s] * validf, mask=wm)
            plsc.store_scatter(obv, [s4 + 2], y2v[s] * validf, mask=wm)
            plsc.store_scatter(obv, [s4 + 3], x2v[s] * validf, mask=wm)
            return base + jnp.sum(kbi)

        lax.fori_loop(0, NCHUNK, comp_body, 0)

        pltpu.sync_copy(obv, rois_hbm.at[w])
        pltpu.sync_copy(osv, rsc_hbm.at[w])


def kernel(rpn_bbox_deltas, rpn_probs, anchors):
    deltas = rpn_bbox_deltas.reshape(B, A, 4)
    probs = rpn_probs.reshape(B, A)
    dts = jnp.transpose(deltas, (0, 2, 1))
    dts = jnp.pad(dts, ((0, 0), (0, 0), (0, AP - A))).reshape(B * 4, AP)
    anc = jnp.pad(anchors.T, ((0, 0), (0, AP - A)))
    ps, pi = lax.top_k(probs, N)
    rois, rsc = _sc_nms(dts, anc, ps, pi.astype(jnp.int32))
    return rois.reshape(B, NO, 4), rsc[:, :NO]


# speedup vs baseline: 11.9416x; 1.0100x over previous
import functools

import jax
import jax.numpy as jnp
from jax import lax
from jax.experimental import pallas as pl
from jax.experimental.pallas import tpu as pltpu
from jax.experimental.pallas import tpu_sc as plsc

B = 4
A = 8649
AP = 8704
N = 6000
NO = 1500
NOP = 1504
L = 16
NCHUNK = N // L
IOU_T = 0.7
V0, V1, V2, V3 = 0.1, 0.1, 0.2, 0.2

SR = 128
SCOL = 128
SORTN = SR * SCOL

_mesh = plsc.VectorSubcoreMesh(
    core_axis_name="c", subcore_axis_name="s", num_cores=2, num_subcores=16)


def _sort_body(prb_ref, skey_ref, sidx_ref):
    key = prb_ref[...]
    I = (lax.broadcasted_iota(jnp.int32, (B, SR, SCOL), 1) * SCOL
         + lax.broadcasted_iota(jnp.int32, (B, SR, SCOL), 2))
    idx = I
    for p in range(14):
        for q in range(p, -1, -1):
            s = 1 << q
            if s < SCOL:
                axis, sh = 2, s
            else:
                axis, sh = 1, s // SCOL
            ok_m = jnp.roll(key, -sh, axis=axis)
            ok_p = jnp.roll(key, sh, axis=axis)
            oi_m = jnp.roll(idx, -sh, axis=axis)
            oi_p = jnp.roll(idx, sh, axis=axis)
            up = (I & s) == 0
            ok = jnp.where(up, ok_m, ok_p)
            oi = jnp.where(up, oi_m, oi_p)
            desc = ((I >> (p + 1)) & 1) == 0
            sg = (key > ok) | ((key == ok) & (idx < oi))
            pref = sg ^ (up == desc)
            key = jnp.where(pref, ok, key)
            idx = jnp.where(pref, oi, idx)
    skey_ref[...] = key
    sidx_ref[...] = idx


_tc_sort = pl.pallas_call(
    _sort_body,
    out_shape=(jax.ShapeDtypeStruct((B, SR, SCOL), jnp.float32),
               jax.ShapeDtypeStruct((B, SR, SCOL), jnp.int32)),
)


@functools.partial(
    pl.kernel,
    out_type=(
        jax.ShapeDtypeStruct((B, N), jnp.float32),
        jax.ShapeDtypeStruct((B, NOP), jnp.float32),
    ),
    mesh=_mesh,
    compiler_params=pltpu.CompilerParams(needs_layout_passes=False),
    scratch_types=[
        pltpu.VMEM((AP,), jnp.float32),
        pltpu.VMEM((AP,), jnp.float32),
        pltpu.VMEM((AP,), jnp.float32),
        pltpu.VMEM((AP,), jnp.float32),
        pltpu.VMEM((AP,), jnp.float32),
        pltpu.VMEM((AP,), jnp.float32),
        pltpu.VMEM((AP,), jnp.float32),
        pltpu.VMEM((AP,), jnp.float32),
        pltpu.VMEM((N,), jnp.int32),
        pltpu.VMEM((N,), jnp.float32),
        pltpu.VMEM((N,), jnp.float32),
        pltpu.VMEM((N,), jnp.float32),
        pltpu.VMEM((N,), jnp.float32),
        pltpu.VMEM((N,), jnp.float32),
        pltpu.VMEM((N,), jnp.float32),
        pltpu.VMEM((N,), jnp.int32),
        pltpu.VMEM((N,), jnp.float32),
        pltpu.VMEM((NOP,), jnp.float32),
    ],
)
def _sc_nms(dts_hbm, anc_hbm, sc_hbm, idx_hbm, rois_hbm, rsc_hbm,
            d0v, d1v, d2v, d3v, a0v, a1v, a2v, a3v,
            iv, sv, y1v, x1v, y2v, x2v, arv, kv, obv, osv):
    w = lax.axis_index("s") * 2 + lax.axis_index("c")

    @pl.when(w < B)
    def _body():
        pltpu.sync_copy(dts_hbm.at[w * 4], d0v)
        pltpu.sync_copy(dts_hbm.at[w * 4 + 1], d1v)
        pltpu.sync_copy(dts_hbm.at[w * 4 + 2], d2v)
        pltpu.sync_copy(dts_hbm.at[w * 4 + 3], d3v)
        pltpu.sync_copy(anc_hbm.at[0], a0v)
        pltpu.sync_copy(anc_hbm.at[1], a1v)
        pltpu.sync_copy(anc_hbm.at[2], a2v)
        pltpu.sync_copy(anc_hbm.at[3], a3v)
        pltpu.sync_copy(sc_hbm.at[w], sv)
        pltpu.sync_copy(idx_hbm.at[w], iv)

        lane = lax.iota(jnp.int32, L)

        def gather_body(ci, carry):
            s = pl.ds(ci * L, L)
            ix = iv[s]
            d0 = plsc.load_gather(d0v, [ix])
            d1 = plsc.load_gather(d1v, [ix])
            d2 = plsc.load_gather(d2v, [ix])
            d3 = plsc.load_gather(d3v, [ix])
            ay1 = plsc.load_gather(a0v, [ix])
            ax1 = plsc.load_gather(a1v, [ix])
            ay2 = plsc.load_gather(a2v, [ix])
            ax2 = plsc.load_gather(a3v, [ix])
            aw = ax2 - ax1
            ah = ay2 - ay1
            acx = ax1 + 0.5 * aw
            acy = ay1 + 0.5 * ah
            bw = jnp.exp(d3 * V3) * aw
            bh = jnp.exp(d2 * V2) * ah
            bcx = d1 * V1 * aw + acx
            bcy = d0 * V0 * ah + acy
            yy1 = jnp.clip(bcy - 0.5 * bh, 0.0, 1.0)
            xx1 = jnp.clip(bcx - 0.5 * bw, 0.0, 1.0)
            yy2 = jnp.clip(bh + (bcy - 0.5 * bh), 0.0, 1.0)
            xx2 = jnp.clip(bw + (bcx - 0.5 * bw), 0.0, 1.0)
            y1v[s] = yy1
            x1v[s] = xx1
            y2v[s] = yy2
            x2v[s] = xx2
            arv[s] = (yy2 - yy1) * (xx2 - xx1)
            kv[s] = jnp.ones((L,), jnp.int32)
            return carry

        lax.fori_loop(0, NCHUNK, gather_body, 0)

        def row_body(i, carry):
            spi = jnp.full((L,), i, jnp.int32)
            alive = plsc.load_gather(kv, [spi])

            @pl.when(alive[0] == 1)
            def _row():
                by1 = plsc.load_gather(y1v, [spi])
                bx1 = plsc.load_gather(x1v, [spi])
                by2 = plsc.load_gather(y2v, [spi])
                bx2 = plsc.load_gather(x2v, [spi])
                bar = plsc.load_gather(arv, [spi])

                def chunk_body(c, carry2):
                    s = pl.ds(c * L, L)
                    iy1 = jnp.maximum(y1v[s], by1)
                    ix1 = jnp.maximum(x1v[s], bx1)
                    iy2 = jnp.minimum(y2v[s], by2)
                    ix2 = jnp.minimum(x2v[s], bx2)
                    inter = (jnp.maximum(iy2 - iy1, 0.0)
                             * jnp.maximum(ix2 - ix1, 0.0))
                    denom = arv[s] + bar - inter + 1e-9
                    jidx = c * L + lane
                    supp = (inter > IOU_T * denom) & (jidx > i)
                    kv[s] = jnp.where(supp, 0, kv[s])
                    return carry2

                lax.fori_loop(i // L, NCHUNK, chunk_body, 0)
            return carry

        lax.fori_loop(0, N, row_body, 0)

        def zero_body(ci, carry):
            obv[pl.ds(ci * L, L)] = jnp.zeros((L,), jnp.float32)
            return carry

        lax.fori_loop(0, NCHUNK, zero_body, 0)

        def zero2_body(ci, carry):
            osv[pl.ds(ci * L, L)] = jnp.zeros((L,), jnp.float32)
            return carry

        lax.fori_loop(0, NOP // L, zero2_body, 0)

        def comp_body(ci, base):
            s = pl.ds(ci * L, L)
            kb = kv[s] == 1
            kbi = jnp.where(kb, 1, 0)
            pos = plsc.cumsum(kbi)
            slot = base + pos - 1
            wm = kb & (slot < NO)
            scs = sv[s]
            validf = jnp.where(scs > 0.0, 1.0, 0.0)
            plsc.store_scatter(osv, [slot], scs * validf, mask=wm)
            s4 = slot * 4
            plsc.store_scatter(obv, [s4], y1v[s] * validf, mask=wm)
            plsc.store_scatter(obv, [s4 + 1], x1v[s] * validf, mask=wm)
            plsc.store_scatter(obv, [s4 + 2], y2v[s] * validf, mask=wm)
            plsc.store_scatter(obv, [s4 + 3], x2v[s] * validf, mask=wm)
            return base + jnp.sum(kbi)

        lax.fori_loop(0, NCHUNK, comp_body, 0)

        pltpu.sync_copy(obv, rois_hbm.at[w])
        pltpu.sync_copy(osv, rsc_hbm.at[w])


def kernel(rpn_bbox_deltas, rpn_probs, anchors):
    deltas = rpn_bbox_deltas.reshape(B, A, 4)
    probs = rpn_probs.reshape(B, A)
    dts = jnp.transpose(deltas, (0, 2, 1))
    dts = jnp.pad(dts, ((0, 0), (0, 0), (0, AP - A))).reshape(B * 4, AP)
    anc = jnp.pad(anchors.T, ((0, 0), (0, AP - A)))
    prbp = jnp.pad(probs, ((0, 0), (0, SORTN - A)), constant_values=-1.0)
    skey, sidx = _tc_sort(prbp.reshape(B, SR, SCOL))
    ps = skey.reshape(B, SORTN)[:, :N]
    pi = sidx.reshape(B, SORTN)[:, :N]
    rois, rsc = _sc_nms(dts, anc, ps, pi)
    return rois.reshape(B, NO, 4), rsc[:, :NO]


# speedup vs baseline: 32.6159x; 2.7313x over previous
import functools

import jax
import jax.numpy as jnp
from jax import lax
from jax.experimental import pallas as pl
from jax.experimental.pallas import tpu as pltpu
from jax.experimental.pallas import tpu_sc as plsc

B = 4
A = 8649
AP = 8704
N = 6000
NO = 1500
NOP = 1504
L = 16
NCHUNK = N // L
IOU_T = 0.7
V0, V1, V2, V3 = 0.1, 0.1, 0.2, 0.2

SR = 128
SCOL = 128
SORTN = SR * SCOL

_mesh = plsc.VectorSubcoreMesh(
    core_axis_name="c", subcore_axis_name="s", num_cores=2, num_subcores=16)


def _sort_body(prb_ref, skey_ref, sidx_ref):
    key = prb_ref[...]
    I = (lax.broadcasted_iota(jnp.int32, (B, SR, SCOL), 1) * SCOL
         + lax.broadcasted_iota(jnp.int32, (B, SR, SCOL), 2))
    idx = I
    for p in range(14):
        for q in range(p, -1, -1):
            s = 1 << q
            if s < SCOL:
                axis, sh = 2, s
            else:
                axis, sh = 1, s // SCOL
            ok_m = jnp.roll(key, -sh, axis=axis)
            ok_p = jnp.roll(key, sh, axis=axis)
            oi_m = jnp.roll(idx, -sh, axis=axis)
            oi_p = jnp.roll(idx, sh, axis=axis)
            up = (I & s) == 0
            ok = jnp.where(up, ok_m, ok_p)
            oi = jnp.where(up, oi_m, oi_p)
            desc = ((I >> (p + 1)) & 1) == 0
            sg = (key > ok) | ((key == ok) & (idx < oi))
            pref = sg ^ (up == desc)
            key = jnp.where(pref, ok, key)
            idx = jnp.where(pref, oi, idx)
    skey_ref[...] = key
    sidx_ref[...] = idx


_tc_sort = pl.pallas_call(
    _sort_body,
    out_shape=(jax.ShapeDtypeStruct((B, SR, SCOL), jnp.float32),
               jax.ShapeDtypeStruct((B, SR, SCOL), jnp.int32)),
)


NP = 6016
S = 752
SCH = S // L
TPI = 8
NPCH = NP // L


@functools.partial(
    pl.kernel,
    out_type=(
        jax.ShapeDtypeStruct((B, N), jnp.float32),
        jax.ShapeDtypeStruct((B, NOP), jnp.float32),
    ),
    mesh=_mesh,
    compiler_params=pltpu.CompilerParams(needs_layout_passes=False),
    scratch_types=[
        pltpu.VMEM((AP,), jnp.float32),
        pltpu.VMEM((AP,), jnp.float32),
        pltpu.VMEM((AP,), jnp.float32),
        pltpu.VMEM((AP,), jnp.float32),
        pltpu.VMEM((AP,), jnp.float32),
        pltpu.VMEM((AP,), jnp.float32),
        pltpu.VMEM((AP,), jnp.float32),
        pltpu.VMEM((AP,), jnp.float32),
        pltpu.VMEM((NP,), jnp.int32),
        pltpu.VMEM((NP,), jnp.float32),
        pltpu.VMEM((NP,), jnp.float32),
        pltpu.VMEM((NP,), jnp.float32),
        pltpu.VMEM((NP,), jnp.float32),
        pltpu.VMEM((NP,), jnp.float32),
        pltpu.VMEM((NP,), jnp.float32),
        pltpu.VMEM((S,), jnp.int32),
        pltpu.VMEM((S,), jnp.int32),
        pltpu.VMEM((N,), jnp.float32),
        pltpu.VMEM((NOP,), jnp.float32),
        pltpu.VMEM_SHARED((B * TPI * S,), jnp.int32),
    ],
)
def _sc_nms(dts_hbm, anc_hbm, sc_hbm, idx_hbm, rois_hbm, rsc_hbm,
            d0v, d1v, d2v, d3v, a0v, a1v, a2v, a3v,
            iv, sv, y1v, x1v, y2v, x2v, arv, kov, pbv, obv, osv, spm):
    c = lax.axis_index("c")
    sid = lax.axis_index("s")
    b = c * 2 + sid // TPI
    t = sid % TPI
    lane = lax.iota(jnp.int32, L)

    pltpu.sync_copy(dts_hbm.at[b * 4], d0v)
    pltpu.sync_copy(dts_hbm.at[b * 4 + 1], d1v)
    pltpu.sync_copy(dts_hbm.at[b * 4 + 2], d2v)
    pltpu.sync_copy(dts_hbm.at[b * 4 + 3], d3v)
    pltpu.sync_copy(anc_hbm.at[0], a0v)
    pltpu.sync_copy(anc_hbm.at[1], a1v)
    pltpu.sync_copy(anc_hbm.at[2], a2v)
    pltpu.sync_copy(anc_hbm.at[3], a3v)
    pltpu.sync_copy(idx_hbm.at[b], iv)

    @pl.when(t == 0)
    def _ldsc():
        pltpu.sync_copy(sc_hbm.at[b], sv)

    def gather_body(ci, carry):
        s = pl.ds(ci * L, L)
        ix = iv[s]
        d0 = plsc.load_gather(d0v, [ix])
        d1 = plsc.load_gather(d1v, [ix])
        d2 = plsc.load_gather(d2v, [ix])
        d3 = plsc.load_gather(d3v, [ix])
        ay1 = plsc.load_gather(a0v, [ix])
        ax1 = plsc.load_gather(a1v, [ix])
        ay2 = plsc.load_gather(a2v, [ix])
        ax2 = plsc.load_gather(a3v, [ix])
        aw = ax2 - ax1
        ah = ay2 - ay1
        acx = ax1 + 0.5 * aw
        acy = ay1 + 0.5 * ah
        bw = jnp.exp(d3 * V3) * aw
        bh = jnp.exp(d2 * V2) * ah
        bcx = d1 * V1 * aw + acx
        bcy = d0 * V0 * ah + acy
        yy1 = jnp.clip(bcy - 0.5 * bh, 0.0, 1.0)
        xx1 = jnp.clip(bcx - 0.5 * bw, 0.0, 1.0)
        yy2 = jnp.clip(bh + (bcy - 0.5 * bh), 0.0, 1.0)
        xx2 = jnp.clip(bw + (bcx - 0.5 * bw), 0.0, 1.0)
        y1v[s] = yy1
        x1v[s] = xx1
        y2v[s] = yy2
        x2v[s] = xx2
        arv[s] = (yy2 - yy1) * (xx2 - xx1)
        return carry

    lax.fori_loop(0, NPCH, gather_body, 0)

    def kinit_body(u, carry):
        gcol = t * S + u * L + lane
        kov[pl.ds(u * L, L)] = jnp.where(gcol < N, 1, 0)
        return carry

    lax.fori_loop(0, SCH, kinit_body, 0)

    def iou_supp(gbase, by1, bx1, by2, bx2, bar):
        s = pl.ds(gbase, L)
        iy1 = jnp.maximum(y1v[s], by1)
        ix1 = jnp.maximum(x1v[s], bx1)
        iy2 = jnp.minimum(y2v[s], by2)
        ix2 = jnp.minimum(x2v[s], bx2)
        inter = jnp.maximum(iy2 - iy1, 0.0) * jnp.maximum(ix2 - ix1, 0.0)
        denom = arv[s] + bar - inter + 1e-9
        return inter > IOU_T * denom

    def round_body(k, carry):
        @pl.when(t == k)
        def _resolve():
            def row_body(r, carry2):
                spl = jnp.full((L,), r, jnp.int32)
                alive = plsc.load_gather(kov, [spl])

                @pl.when(alive[0] == 1)
                def _row():
                    gi = k * S + r
                    spg = jnp.full((L,), gi, jnp.int32)
                    by1 = plsc.load_gather(y1v, [spg])
                    bx1 = plsc.load_gather(x1v, [spg])
                    by2 = plsc.load_gather(y2v, [spg])
                    bx2 = plsc.load_gather(x2v, [spg])
                    bar = plsc.load_gather(arv, [spg])

                    def chunk_body(u, carry3):
                        supp = iou_supp(k * S + u * L, by1, bx1, by2, bx2, bar)
                        supp = supp & (u * L + lane > r)
                        sl = pl.ds(u * L, L)
                        kov[sl] = jnp.where(supp, 0, kov[sl])
                        return carry3

                    lax.fori_loop(r // L, SCH, chunk_body, 0)
                return carry2

            lax.fori_loop(0, S, row_body, 0)
            pltpu.sync_copy(kov, spm.at[pl.ds((b * TPI + k) * S, S)])

        plsc.subcore_barrier()

        @pl.when(t > k)
        def _apply():
            pltpu.sync_copy(spm.at[pl.ds((b * TPI + k) * S, S)], pbv)

            def row_body(r, carry2):
                spl = jnp.full((L,), r, jnp.int32)
                alive = plsc.load_gather(pbv, [spl])

                @pl.when(alive[0] == 1)
                def _row():
                    gi = k * S + r
                    spg = jnp.full((L,), gi, jnp.int32)
                    by1 = plsc.load_gather(y1v, [spg])
                    bx1 = plsc.load_gather(x1v, [spg])
                    by2 = plsc.load_gather(y2v, [spg])
                    bx2 = plsc.load_gather(x2v, [spg])
                    bar = plsc.load_gather(arv, [spg])

                    def chunk_body(u, carry3):
                        supp = iou_supp(t * S + u * L, by1, bx1, by2, bx2, bar)
                        sl = pl.ds(u * L, L)
                        kov[sl] = jnp.where(supp, 0, kov[sl])
                        return carry3

                    lax.fori_loop(0, SCH, chunk_body, 0)
                return carry2

            lax.fori_loop(0, S, row_body, 0)
        return carry

    lax.fori_loop(0, TPI, round_body, 0)
    plsc.subcore_barrier()

    @pl.when(t == 0)
    def _compact():
        def zero_body(ci, carry):
            obv[pl.ds(ci * L, L)] = jnp.zeros((L,), jnp.float32)
            return carry

        lax.fori_loop(0, N // L, zero_body, 0)

        def zero2_body(ci, carry):
            osv[pl.ds(ci * L, L)] = jnp.zeros((L,), jnp.float32)
            return carry

        lax.fori_loop(0, NOP // L, zero2_body, 0)

        def stripe_body(k, base0):
            pltpu.sync_copy(spm.at[pl.ds((b * TPI + k) * S, S)], pbv)

            def comp_body(u, base):
                sl = pl.ds(u * L, L)
                gsl = pl.ds(k * S + u * L, L)
                kb = pbv[sl] == 1
                kbi = jnp.where(kb, 1, 0)
                pos = plsc.cumsum(kbi)
                slot = base + pos - 1
                wm = kb & (slot < NO)
                scs = sv[gsl]
                validf = jnp.where(scs > 0.0, 1.0, 0.0)
                plsc.store_scatter(osv, [slot], scs * validf, mask=wm)
                s4 = slot * 4
                plsc.store_scatter(obv, [s4], y1v[gsl] * validf, mask=wm)
                plsc.store_scatter(obv, [s4 + 1], x1v[gsl] * validf, mask=wm)
                plsc.store_scatter(obv, [s4 + 2], y2v[gsl] * validf, mask=wm)
                plsc.store_scatter(obv, [s4 + 3], x2v[gsl] * validf, mask=wm)
                return base + jnp.sum(kbi)

            return lax.fori_loop(0, SCH, comp_body, base0)

        lax.fori_loop(0, TPI, stripe_body, 0)
        pltpu.sync_copy(obv, rois_hbm.at[b])
        pltpu.sync_copy(osv, rsc_hbm.at[b])


def kernel(rpn_bbox_deltas, rpn_probs, anchors):
    deltas = rpn_bbox_deltas.reshape(B, A, 4)
    probs = rpn_probs.reshape(B, A)
    dts = jnp.transpose(deltas, (0, 2, 1))
    dts = jnp.pad(dts, ((0, 0), (0, 0), (0, AP - A))).reshape(B * 4, AP)
    anc = jnp.pad(anchors.T, ((0, 0), (0, AP - A)))
    prbp = jnp.pad(probs, ((0, 0), (0, SORTN - A)), constant_values=-1.0)
    skey, sidx = _tc_sort(prbp.reshape(B, SR, SCOL))
    ps = jnp.pad(skey.reshape(B, SORTN)[:, :N], ((0, 0), (0, NP - N)))
    pi = jnp.pad(sidx.reshape(B, SORTN)[:, :N], ((0, 0), (0, NP - N)),
                 constant_values=A + 1)
    rois, rsc = _sc_nms(dts, anc, ps, pi)
    return rois.reshape(B, NO, 4), rsc[:, :NO]


# speedup vs baseline: 73.4842x; 2.2530x over previous
import functools

import jax
import jax.numpy as jnp
from jax import lax
from jax.experimental import pallas as pl
from jax.experimental.pallas import tpu as pltpu
from jax.experimental.pallas import tpu_sc as plsc

B = 4
A = 8649
AP = 8704
N = 6000
NO = 1500
NOP = 1504
L = 16
NCHUNK = N // L
IOU_T = 0.7
V0, V1, V2, V3 = 0.1, 0.1, 0.2, 0.2

SR = 128
SCOL = 128
SORTN = SR * SCOL

_mesh = plsc.VectorSubcoreMesh(
    core_axis_name="c", subcore_axis_name="s", num_cores=2, num_subcores=16)


def _sort_body(prb_ref, skey_ref, sidx_ref):
    key = prb_ref[...]
    I = (lax.broadcasted_iota(jnp.int32, (B, SR, SCOL), 1) * SCOL
         + lax.broadcasted_iota(jnp.int32, (B, SR, SCOL), 2))
    idx = I
    for p in range(14):
        for q in range(p, -1, -1):
            s = 1 << q
            if s < SCOL:
                axis, sh = 2, s
            else:
                axis, sh = 1, s // SCOL
            ok_m = jnp.roll(key, -sh, axis=axis)
            ok_p = jnp.roll(key, sh, axis=axis)
            oi_m = jnp.roll(idx, -sh, axis=axis)
            oi_p = jnp.roll(idx, sh, axis=axis)
            up = (I & s) == 0
            ok = jnp.where(up, ok_m, ok_p)
            oi = jnp.where(up, oi_m, oi_p)
            desc = ((I >> (p + 1)) & 1) == 0
            sg = (key > ok) | ((key == ok) & (idx < oi))
            pref = sg ^ (up == desc)
            key = jnp.where(pref, ok, key)
            idx = jnp.where(pref, oi, idx)
    skey_ref[...] = key
    sidx_ref[...] = idx


_tc_sort = pl.pallas_call(
    _sort_body,
    out_shape=(jax.ShapeDtypeStruct((B, SR, SCOL), jnp.float32),
               jax.ShapeDtypeStruct((B, SR, SCOL), jnp.int32)),
)


NP = 6016
S = 752
SCH = S // L
TPI = 8
NPCH = NP // L


@functools.partial(
    pl.kernel,
    out_type=(
        jax.ShapeDtypeStruct((B, N), jnp.float32),
        jax.ShapeDtypeStruct((B, NOP), jnp.float32),
    ),
    mesh=_mesh,
    compiler_params=pltpu.CompilerParams(needs_layout_passes=False),
    scratch_types=[
        pltpu.VMEM((AP,), jnp.float32),
        pltpu.VMEM((AP,), jnp.float32),
        pltpu.VMEM((AP,), jnp.float32),
        pltpu.VMEM((AP,), jnp.float32),
        pltpu.VMEM((AP,), jnp.float32),
        pltpu.VMEM((AP,), jnp.float32),
        pltpu.VMEM((AP,), jnp.float32),
        pltpu.VMEM((AP,), jnp.float32),
        pltpu.VMEM((NP,), jnp.int32),
        pltpu.VMEM((NP,), jnp.float32),
        pltpu.VMEM((NP,), jnp.float32),
        pltpu.VMEM((NP,), jnp.float32),
        pltpu.VMEM((NP,), jnp.float32),
        pltpu.VMEM((NP,), jnp.float32),
        pltpu.VMEM((NP,), jnp.float32),
        pltpu.VMEM((S,), jnp.int32),
        pltpu.VMEM((S,), jnp.int32),
        pltpu.VMEM((N,), jnp.float32),
        pltpu.VMEM((NOP,), jnp.float32),
        pltpu.VMEM_SHARED((B * TPI * S,), jnp.int32),
    ],
)
def _sc_nms(dts_hbm, anc_hbm, sc_hbm, idx_hbm, rois_hbm, rsc_hbm,
            d0v, d1v, d2v, d3v, a0v, a1v, a2v, a3v,
            iv, sv, y1v, x1v, y2v, x2v, arv, kov, pbv, obv, osv, spm):
    c = lax.axis_index("c")
    sid = lax.axis_index("s")
    b = c * 2 + sid // TPI
    t = sid % TPI
    lane = lax.iota(jnp.int32, L)

    pltpu.sync_copy(dts_hbm.at[b * 4], d0v)
    pltpu.sync_copy(dts_hbm.at[b * 4 + 1], d1v)
    pltpu.sync_copy(dts_hbm.at[b * 4 + 2], d2v)
    pltpu.sync_copy(dts_hbm.at[b * 4 + 3], d3v)
    pltpu.sync_copy(anc_hbm.at[0], a0v)
    pltpu.sync_copy(anc_hbm.at[1], a1v)
    pltpu.sync_copy(anc_hbm.at[2], a2v)
    pltpu.sync_copy(anc_hbm.at[3], a3v)
    pltpu.sync_copy(idx_hbm.at[b], iv)

    @pl.when(t == 0)
    def _ldsc():
        pltpu.sync_copy(sc_hbm.at[b], sv)

    def gather_body(ci, carry):
        s = pl.ds(ci * L, L)
        ix = iv[s]
        d0 = plsc.load_gather(d0v, [ix])
        d1 = plsc.load_gather(d1v, [ix])
        d2 = plsc.load_gather(d2v, [ix])
        d3 = plsc.load_gather(d3v, [ix])
        ay1 = plsc.load_gather(a0v, [ix])
        ax1 = plsc.load_gather(a1v, [ix])
        ay2 = plsc.load_gather(a2v, [ix])
        ax2 = plsc.load_gather(a3v, [ix])
        aw = ax2 - ax1
        ah = ay2 - ay1
        acx = ax1 + 0.5 * aw
        acy = ay1 + 0.5 * ah
        bw = jnp.exp(d3 * V3) * aw
        bh = jnp.exp(d2 * V2) * ah
        bcx = d1 * V1 * aw + acx
        bcy = d0 * V0 * ah + acy
        yy1 = jnp.clip(bcy - 0.5 * bh, 0.0, 1.0)
        xx1 = jnp.clip(bcx - 0.5 * bw, 0.0, 1.0)
        yy2 = jnp.clip(bh + (bcy - 0.5 * bh), 0.0, 1.0)
        xx2 = jnp.clip(bw + (bcx - 0.5 * bw), 0.0, 1.0)
        y1v[s] = yy1
        x1v[s] = xx1
        y2v[s] = yy2
        x2v[s] = xx2
        arv[s] = (yy2 - yy1) * (xx2 - xx1)
        return carry

    lax.fori_loop(0, NPCH, gather_body, 0)

    def kinit_body(u, carry):
        gcol = t * S + u * L + lane
        kov[pl.ds(u * L, L)] = jnp.where(gcol < N, 0, 1)
        return carry

    lax.fori_loop(0, SCH, kinit_body, 0)

    def iou_supp(gbase, by1, bx1, by2, bx2, bar):
        s = pl.ds(gbase, L)
        iy1 = jnp.maximum(y1v[s], by1)
        ix1 = jnp.maximum(x1v[s], bx1)
        iy2 = jnp.minimum(y2v[s], by2)
        ix2 = jnp.minimum(x2v[s], bx2)
        inter = jnp.maximum(iy2 - iy1, 0.0) * jnp.maximum(ix2 - ix1, 0.0)
        denom = arv[s] + bar - inter + 1e-9
        return inter > IOU_T * denom

    def round_body(k, carry):
        @pl.when(t == k)
        def _resolve():
            def row_body(r, carry2):
                spl = jnp.full((L,), r, jnp.int32)
                alive = plsc.load_gather(kov, [spl])

                @pl.when(alive[0] == 0)
                def _row():
                    gi = k * S + r
                    spg = jnp.full((L,), gi, jnp.int32)
                    by1 = plsc.load_gather(y1v, [spg])
                    bx1 = plsc.load_gather(x1v, [spg])
                    by2 = plsc.load_gather(y2v, [spg])
                    bx2 = plsc.load_gather(x2v, [spg])
                    bar = plsc.load_gather(arv, [spg])

                    ud = r // L
                    supp0 = iou_supp(k * S + ud * L, by1, bx1, by2, bx2, bar)
                    supp0 = supp0 & (ud * L + lane > r)
                    plsc.addupdate(kov.at[pl.ds(ud * L, L)],
                                   jnp.where(supp0, 1, 0))

                    @plsc.parallel_loop(ud + 1, SCH, unroll=2)
                    def chunk_body(u):
                        supp = iou_supp(k * S + u * L, by1, bx1, by2, bx2, bar)
                        plsc.addupdate(kov.at[pl.ds(u * L, L)],
                                       jnp.where(supp, 1, 0))
                return carry2

            lax.fori_loop(0, S, row_body, 0)
            pltpu.sync_copy(kov, spm.at[pl.ds((b * TPI + k) * S, S)])

        plsc.subcore_barrier()

        @pl.when(t > k)
        def _apply():
            pltpu.sync_copy(spm.at[pl.ds((b * TPI + k) * S, S)], pbv)

            def row_body(r, carry2):
                spl = jnp.full((L,), r, jnp.int32)
                alive = plsc.load_gather(pbv, [spl])

                @pl.when(alive[0] == 0)
                def _row():
                    gi = k * S + r
                    spg = jnp.full((L,), gi, jnp.int32)
                    by1 = plsc.load_gather(y1v, [spg])
                    bx1 = plsc.load_gather(x1v, [spg])
                    by2 = plsc.load_gather(y2v, [spg])
                    bx2 = plsc.load_gather(x2v, [spg])
                    bar = plsc.load_gather(arv, [spg])

                    @plsc.parallel_loop(0, SCH, unroll=2)
                    def chunk_body(u):
                        supp = iou_supp(t * S + u * L, by1, bx1, by2, bx2, bar)
                        plsc.addupdate(kov.at[pl.ds(u * L, L)],
                                       jnp.where(supp, 1, 0))
                return carry2

            lax.fori_loop(0, S, row_body, 0)
        return carry

    lax.fori_loop(0, TPI, round_body, 0)
    plsc.subcore_barrier()

    @pl.when(t == 0)
    def _compact():
        def zero_body(ci, carry):
            obv[pl.ds(ci * L, L)] = jnp.zeros((L,), jnp.float32)
            return carry

        lax.fori_loop(0, N // L, zero_body, 0)

        def zero2_body(ci, carry):
            osv[pl.ds(ci * L, L)] = jnp.zeros((L,), jnp.float32)
            return carry

        lax.fori_loop(0, NOP // L, zero2_body, 0)

        def stripe_body(k, base0):
            pltpu.sync_copy(spm.at[pl.ds((b * TPI + k) * S, S)], pbv)

            def comp_body(u, base):
                sl = pl.ds(u * L, L)
                gsl = pl.ds(k * S + u * L, L)
                kb = pbv[sl] == 0
                kbi = jnp.where(kb, 1, 0)
                pos = plsc.cumsum(kbi)
                slot = base + pos - 1
                wm = kb & (slot < NO)
                scs = sv[gsl]
                validf = jnp.where(scs > 0.0, 1.0, 0.0)
                plsc.store_scatter(osv, [slot], scs * validf, mask=wm)
                s4 = slot * 4
                plsc.store_scatter(obv, [s4], y1v[gsl] * validf, mask=wm)
                plsc.store_scatter(obv, [s4 + 1], x1v[gsl] * validf, mask=wm)
                plsc.store_scatter(obv, [s4 + 2], y2v[gsl] * validf, mask=wm)
                plsc.store_scatter(obv, [s4 + 3], x2v[gsl] * validf, mask=wm)
                return base + jnp.sum(kbi)

            return lax.fori_loop(0, SCH, comp_body, base0)

        lax.fori_loop(0, TPI, stripe_body, 0)
        pltpu.sync_copy(obv, rois_hbm.at[b])
        pltpu.sync_copy(osv, rsc_hbm.at[b])


def kernel(rpn_bbox_deltas, rpn_probs, anchors):
    deltas = rpn_bbox_deltas.reshape(B, A, 4)
    probs = rpn_probs.reshape(B, A)
    dts = jnp.transpose(deltas, (0, 2, 1))
    dts = jnp.pad(dts, ((0, 0), (0, 0), (0, AP - A))).reshape(B * 4, AP)
    anc = jnp.pad(anchors.T, ((0, 0), (0, AP - A)))
    prbp = jnp.pad(probs, ((0, 0), (0, SORTN - A)), constant_values=-1.0)
    skey, sidx = _tc_sort(prbp.reshape(B, SR, SCOL))
    ps = jnp.pad(skey.reshape(B, SORTN)[:, :N], ((0, 0), (0, NP - N)))
    pi = jnp.pad(sidx.reshape(B, SORTN)[:, :N], ((0, 0), (0, NP - N)),
                 constant_values=A + 1)
    rois, rsc = _sc_nms(dts, anc, ps, pi)
    return rois.reshape(B, NO, 4), rsc[:, :NO]
